# Initial kernel scaffold; baseline (speedup 1.0000x reference)
import jax
import jax.numpy as jnp
from jax.experimental import pallas as pl


def kernel(x, edge_index, W1a, b1a, W1b, b1b, g1, beta1, W2a, b2a, W2b, b2b, g2, beta2):
    raise NotImplementedError("write your pallas kernel here")


# speedup vs baseline: 2.9540x; 2.9540x over previous
import functools

import jax
import jax.numpy as jnp
from jax import lax
from jax.experimental import pallas as pl
from jax.experimental.pallas import tpu as pltpu
from jax.experimental.pallas import tpu_sc as plsc

N = 10000
E = 160000
COLS = 256
HALF = 128
OD = 2
NT = 16
CHUNK = 128
KCH = 80
E_PAD = NT * KCH * CHUNK
N_PAD = 10240
RPT = N_PAD // NT
RPT_LAST = N - (NT - 1) * RPT
BN_EPS = 1e-5
BM = 1000

def _mesh():
    return plsc.VectorSubcoreMesh(core_axis_name="c", subcore_axis_name="s")


def _seg_body(xlo, xhi, src3, dst3, zlo, zhi, srcv, dstv, rows, aggsh):
    c = lax.axis_index("c")
    s = lax.axis_index("s")
    base = s * RPT
    last = (NT - 1) * RPT

    def run(xh, zh):
        @pl.when(s < NT - 1)
        def _():
            pltpu.sync_copy(xh.at[pl.ds(base, RPT)], aggsh.at[pl.ds(base, RPT)])

        @pl.when(s == NT - 1)
        def _():
            pltpu.sync_copy(xh.at[pl.ds(last, RPT_LAST)],
                            aggsh.at[pl.ds(last, RPT_LAST)])

        plsc.subcore_barrier()

        pltpu.sync_copy(src3.at[s], srcv)
        pltpu.sync_copy(dst3.at[s], dstv)

        @pl.loop(0, KCH)
        def _(k):
            pltpu.sync_copy(xh.at[srcv.at[k]], rows)
            pltpu.sync_copy(rows, aggsh.at[dstv.at[k]], add=True)

        plsc.subcore_barrier()

        @pl.when(s < NT - 1)
        def _():
            pltpu.sync_copy(aggsh.at[pl.ds(base, RPT)], zh.at[pl.ds(base, RPT)])

        @pl.when(s == NT - 1)
        def _():
            pltpu.sync_copy(aggsh.at[pl.ds(last, RPT_LAST)],
                            zh.at[pl.ds(last, RPT_LAST)])

    @pl.when(c == 0)
    def _():
        run(xlo, zlo)

    @pl.when(c == 1)
    def _():
        run(xhi, zhi)


def _segment_sc(xlo, xhi, src3, dst3):
    f = pl.kernel(
        _seg_body,
        out_type=[jax.ShapeDtypeStruct((N, HALF), jnp.float32),
                  jax.ShapeDtypeStruct((N, HALF), jnp.float32)],
        mesh=_mesh(),
        scratch_types=[
            pltpu.VMEM((KCH, CHUNK), jnp.int32),
            pltpu.VMEM((KCH, CHUNK), jnp.int32),
            pltpu.VMEM((CHUNK, HALF), jnp.float32),
            pltpu.VMEM_SHARED((N_PAD, HALF), jnp.float32),
        ],
    )
    return f(xlo, xhi, src3, dst3)


def _dot(a, b):
    return jnp.dot(a, b, preferred_element_type=jnp.float32,
                   precision=lax.Precision.DEFAULT)


def _mlp_body(zlo, zhi, wa, ba, wb, bb, h_out, stats):
    i = pl.program_id(0)
    u = _dot(zlo[...], wa[0:HALF, :]) + _dot(zhi[...], wa[HALF:COLS, :])
    u = jnp.maximum(u + ba[...], 0.0)
    h = _dot(u, wb[...]) + bb[...]
    h = jnp.maximum(h, 0.0)
    h_out[...] = h

    @pl.when(i == 0)
    def _():
        stats[...] = jnp.zeros_like(stats)

    stats[0:1, :] += jnp.sum(h, axis=0, keepdims=True)
    stats[1:2, :] += jnp.sum(h * h, axis=0, keepdims=True)


def _mlp(zlo, zhi, wa, ba, wb, bb, wout):
    return pl.pallas_call(
        _mlp_body,
        grid=(N // BM,),
        in_specs=[
            pl.BlockSpec((BM, HALF), lambda i: (i, 0)),
            pl.BlockSpec((BM, HALF), lambda i: (i, 0)),
            pl.BlockSpec((COLS, COLS), lambda i: (0, 0)),
            pl.BlockSpec((1, COLS), lambda i: (0, 0)),
            pl.BlockSpec((COLS, wout), lambda i: (0, 0)),
            pl.BlockSpec((1, wout), lambda i: (0, 0)),
        ],
        out_specs=[
            pl.BlockSpec((BM, wout), lambda i: (i, 0)),
            pl.BlockSpec((8, wout), lambda i: (0, 0)),
        ],
        out_shape=[jax.ShapeDtypeStruct((N, wout), jnp.float32),
                   jax.ShapeDtypeStruct((8, wout), jnp.float32)],
    )(zlo, zhi, wa, ba, wb, bb)


def _norm_split_body(h, sref, tref, olo, ohi):
    v = h[...] * sref[...] + tref[...]
    olo[...] = v[:, 0:HALF]
    ohi[...] = v[:, HALF:COLS]


def _norm_split(h, sv, tv):
    return pl.pallas_call(
        _norm_split_body,
        grid=(N // BM,),
        in_specs=[
            pl.BlockSpec((BM, COLS), lambda i: (i, 0)),
            pl.BlockSpec((1, COLS), lambda i: (0, 0)),
            pl.BlockSpec((1, COLS), lambda i: (0, 0)),
        ],
        out_specs=[
            pl.BlockSpec((BM, HALF), lambda i: (i, 0)),
            pl.BlockSpec((BM, HALF), lambda i: (i, 0)),
        ],
        out_shape=[jax.ShapeDtypeStruct((N, HALF), jnp.float32),
                   jax.ShapeDtypeStruct((N, HALF), jnp.float32)],
    )(h, sv, tv)


def _norm_body(h, sref, tref, out):
    out[...] = h[...] * sref[...] + tref[...]


def _norm(h, sv, tv):
    return pl.pallas_call(
        _norm_body,
        grid=(N // BM,),
        in_specs=[
            pl.BlockSpec((BM, HALF), lambda i: (i, 0)),
            pl.BlockSpec((1, HALF), lambda i: (0, 0)),
            pl.BlockSpec((1, HALF), lambda i: (0, 0)),
        ],
        out_specs=pl.BlockSpec((BM, HALF), lambda i: (i, 0)),
        out_shape=jax.ShapeDtypeStruct((N, HALF), jnp.float32),
    )(h, sv, tv)


def _bn_affine(stats, gamma, beta):
    m = stats[0] / N
    v = stats[1] / N - m * m
    s = gamma * lax.rsqrt(v + BN_EPS)
    t = beta - m * s
    return s[None, :], t[None, :]


def kernel(x, edge_index, W1a, b1a, W1b, b1b, g1, beta1,
           W2a, b2a, W2b, b2b, g2, beta2):
    x = x.astype(jnp.float32)
    src = edge_index[0].astype(jnp.int32)
    dst = edge_index[1].astype(jnp.int32)
    pad = E_PAD - E
    src3 = jnp.concatenate([src, jnp.zeros((pad,), jnp.int32)]).reshape(
        NT, KCH, CHUNK)
    dst3 = jnp.concatenate([dst, jnp.full((pad,), N, jnp.int32)]).reshape(
        NT, KCH, CHUNK)

    xlo = x[:, :HALF]
    xhi = x[:, HALF:]

    z1lo, z1hi = _segment_sc(xlo, xhi, src3, dst3)
    h1, st1 = _mlp(z1lo, z1hi, W1a, b1a[None, :], W1b, b1b[None, :], COLS)
    s1, t1 = _bn_affine(st1, g1, beta1)
    h1lo, h1hi = _norm_split(h1, s1, t1)

    z2lo, z2hi = _segment_sc(h1lo, h1hi, src3, dst3)
    w2bp = jnp.pad(W2b, ((0, 0), (0, HALF - OD)))
    b2bp = jnp.pad(b2b, (0, HALF - OD))
    o, st2 = _mlp(z2lo, z2hi, W2a, b2a[None, :], w2bp, b2bp[None, :], HALF)
    g2p = jnp.pad(g2, (0, HALF - OD))
    beta2p = jnp.pad(beta2, (0, HALF - OD))
    s2, t2 = _bn_affine(st2, g2p, beta2p)
    outp = _norm(o, s2, t2)
    return outp[:, :OD]


# speedup vs baseline: 3.1070x; 1.0518x over previous
import functools

import jax
import jax.numpy as jnp
from jax import lax
from jax.experimental import pallas as pl
from jax.experimental.pallas import tpu as pltpu
from jax.experimental.pallas import tpu_sc as plsc

N = 10000
E = 160000
COLS = 256
HALF = 128
OD = 2
NT = 16
CHUNK = 64
KCH = 160
E_PAD = NT * KCH * CHUNK
N_PAD = 10008
RPT = 632
RPT_LAST = N - (NT - 1) * RPT
BN_EPS = 1e-5
BM = 1000

def _mesh():
    return plsc.VectorSubcoreMesh(core_axis_name="c", subcore_axis_name="s")


NBUF = 2
HKCH = KCH // 2
HROUNDS = HKCH // NBUF


def _seg_body(xlo, xhi, src3, dst3, zlo, zhi, srcv, dstv, rows, aggsh, *sems):
    c = lax.axis_index("c")
    s = lax.axis_index("s")
    base = s * RPT
    last = (NT - 1) * RPT
    gsems = sems[:NBUF]
    ssems = sems[NBUF:]

    def run(xh, zh):
        @pl.when(s < NT - 1)
        def _():
            pltpu.sync_copy(xh.at[pl.ds(base, RPT)], aggsh.at[pl.ds(base, RPT)])

        @pl.when(s == NT - 1)
        def _():
            pltpu.sync_copy(xh.at[pl.ds(last, RPT_LAST)],
                            aggsh.at[pl.ds(last, RPT_LAST)])

        plsc.subcore_barrier()

        for hf in range(2):
            pltpu.sync_copy(src3.at[s].at[pl.ds(hf * HKCH, HKCH)], srcv)
            pltpu.sync_copy(dst3.at[s].at[pl.ds(hf * HKCH, HKCH)], dstv)

            for b in range(NBUF):
                pltpu.async_copy(xh.at[srcv.at[b]], rows.at[b], gsems[b])

            @pl.loop(0, HROUNDS)
            def _(j):
                c0 = j * NBUF
                for b in range(NBUF):
                    ce = c0 + b
                    pltpu.make_async_copy(xh.at[srcv.at[ce]], rows.at[b],
                                          gsems[b]).wait()
                    pltpu.async_copy(rows.at[b], aggsh.at[dstv.at[ce]],
                                     ssems[b], add=True)

                @pl.when(j < HROUNDS - 1)
                def _():
                    for b in range(NBUF):
                        ce = c0 + b
                        pltpu.make_async_copy(rows.at[b],
                                              aggsh.at[dstv.at[ce]],
                                              ssems[b]).wait()
                        pltpu.async_copy(xh.at[srcv.at[ce + NBUF]], rows.at[b],
                                         gsems[b])

            for b in range(NBUF):
                ce = HKCH - NBUF + b
                pltpu.make_async_copy(rows.at[b], aggsh.at[dstv.at[ce]],
                                      ssems[b]).wait()

        plsc.subcore_barrier()

        @pl.when(s < NT - 1)
        def _():
            pltpu.sync_copy(aggsh.at[pl.ds(base, RPT)], zh.at[pl.ds(base, RPT)])

        @pl.when(s == NT - 1)
        def _():
            pltpu.sync_copy(aggsh.at[pl.ds(last, RPT_LAST)],
                            zh.at[pl.ds(last, RPT_LAST)])

    @pl.when(c == 0)
    def _():
        run(xlo, zlo)

    @pl.when(c == 1)
    def _():
        run(xhi, zhi)


def _segment_sc(xlo, xhi, src3, dst3):
    f = pl.kernel(
        _seg_body,
        out_type=[jax.ShapeDtypeStruct((N, HALF), jnp.float32),
                  jax.ShapeDtypeStruct((N, HALF), jnp.float32)],
        mesh=_mesh(),
        scratch_types=[
            pltpu.VMEM((HKCH, CHUNK), jnp.int32),
            pltpu.VMEM((HKCH, CHUNK), jnp.int32),
            pltpu.VMEM((NBUF, CHUNK, HALF), jnp.float32),
            pltpu.VMEM_SHARED((N_PAD, HALF), jnp.float32),
        ] + [pltpu.SemaphoreType.DMA] * (2 * NBUF),
    )
    return f(xlo, xhi, src3, dst3)


def _dot(a, b):
    return jnp.dot(a, b, preferred_element_type=jnp.float32,
                   precision=lax.Precision.DEFAULT)


def _mlp_body(zlo, zhi, wa, ba, wb, bb, h_out, stats):
    i = pl.program_id(0)
    u = _dot(zlo[...], wa[0:HALF, :]) + _dot(zhi[...], wa[HALF:COLS, :])
    u = jnp.maximum(u + ba[...], 0.0)
    h = _dot(u, wb[...]) + bb[...]
    h = jnp.maximum(h, 0.0)
    h_out[...] = h

    @pl.when(i == 0)
    def _():
        stats[...] = jnp.zeros_like(stats)

    stats[0:1, :] += jnp.sum(h, axis=0, keepdims=True)
    stats[1:2, :] += jnp.sum(h * h, axis=0, keepdims=True)


def _mlp(zlo, zhi, wa, ba, wb, bb, wout):
    return pl.pallas_call(
        _mlp_body,
        grid=(N // BM,),
        in_specs=[
            pl.BlockSpec((BM, HALF), lambda i: (i, 0)),
            pl.BlockSpec((BM, HALF), lambda i: (i, 0)),
            pl.BlockSpec((COLS, COLS), lambda i: (0, 0)),
            pl.BlockSpec((1, COLS), lambda i: (0, 0)),
            pl.BlockSpec((COLS, wout), lambda i: (0, 0)),
            pl.BlockSpec((1, wout), lambda i: (0, 0)),
        ],
        out_specs=[
            pl.BlockSpec((BM, wout), lambda i: (i, 0)),
            pl.BlockSpec((8, wout), lambda i: (0, 0)),
        ],
        out_shape=[jax.ShapeDtypeStruct((N, wout), jnp.float32),
                   jax.ShapeDtypeStruct((8, wout), jnp.float32)],
    )(zlo, zhi, wa, ba, wb, bb)


def _norm_split_body(h, sref, tref, olo, ohi):
    v = h[...] * sref[...] + tref[...]
    olo[...] = v[:, 0:HALF]
    ohi[...] = v[:, HALF:COLS]


def _norm_split(h, sv, tv):
    return pl.pallas_call(
        _norm_split_body,
        grid=(N // BM,),
        in_specs=[
            pl.BlockSpec((BM, COLS), lambda i: (i, 0)),
            pl.BlockSpec((1, COLS), lambda i: (0, 0)),
            pl.BlockSpec((1, COLS), lambda i: (0, 0)),
        ],
        out_specs=[
            pl.BlockSpec((BM, HALF), lambda i: (i, 0)),
            pl.BlockSpec((BM, HALF), lambda i: (i, 0)),
        ],
        out_shape=[jax.ShapeDtypeStruct((N, HALF), jnp.float32),
                   jax.ShapeDtypeStruct((N, HALF), jnp.float32)],
    )(h, sv, tv)


def _norm_body(h, sref, tref, out):
    out[...] = h[...] * sref[...] + tref[...]


def _norm(h, sv, tv):
    return pl.pallas_call(
        _norm_body,
        grid=(N // BM,),
        in_specs=[
            pl.BlockSpec((BM, HALF), lambda i: (i, 0)),
            pl.BlockSpec((1, HALF), lambda i: (0, 0)),
            pl.BlockSpec((1, HALF), lambda i: (0, 0)),
        ],
        out_specs=pl.BlockSpec((BM, HALF), lambda i: (i, 0)),
        out_shape=jax.ShapeDtypeStruct((N, HALF), jnp.float32),
    )(h, sv, tv)


def _bn_affine(stats, gamma, beta):
    m = stats[0] / N
    v = stats[1] / N - m * m
    s = gamma * lax.rsqrt(v + BN_EPS)
    t = beta - m * s
    return s[None, :], t[None, :]


def kernel(x, edge_index, W1a, b1a, W1b, b1b, g1, beta1,
           W2a, b2a, W2b, b2b, g2, beta2):
    x = x.astype(jnp.float32)
    src = edge_index[0].astype(jnp.int32)
    dst = edge_index[1].astype(jnp.int32)
    pad = E_PAD - E
    src3 = jnp.concatenate([src, jnp.zeros((pad,), jnp.int32)]).reshape(
        NT, KCH, CHUNK)
    dst3 = jnp.concatenate([dst, jnp.full((pad,), N, jnp.int32)]).reshape(
        NT, KCH, CHUNK)

    xlo = x[:, :HALF]
    xhi = x[:, HALF:]

    z1lo, z1hi = _segment_sc(xlo, xhi, src3, dst3)
    h1, st1 = _mlp(z1lo, z1hi, W1a, b1a[None, :], W1b, b1b[None, :], COLS)
    s1, t1 = _bn_affine(st1, g1, beta1)
    h1lo, h1hi = _norm_split(h1, s1, t1)

    z2lo, z2hi = _segment_sc(h1lo, h1hi, src3, dst3)
    w2bp = jnp.pad(W2b, ((0, 0), (0, HALF - OD)))
    b2bp = jnp.pad(b2b, (0, HALF - OD))
    o, st2 = _mlp(z2lo, z2hi, W2a, b2a[None, :], w2bp, b2bp[None, :], HALF)
    g2p = jnp.pad(g2, (0, HALF - OD))
    beta2p = jnp.pad(beta2, (0, HALF - OD))
    s2, t2 = _bn_affine(st2, g2p, beta2p)
    outp = _norm(o, s2, t2)
    return outp[:, :OD]


# speedup vs baseline: 3.2309x; 1.0399x over previous
import functools

import jax
import jax.numpy as jnp
from jax import lax
from jax.experimental import pallas as pl
from jax.experimental.pallas import tpu as pltpu
from jax.experimental.pallas import tpu_sc as plsc

N = 10000
E = 160000
COLS = 256
HALF = 128
OD = 2
NT = 16
CHUNK = 128
KCH = 80
E_PAD = NT * KCH * CHUNK
N_PAD = 10008
RPT = 632
RPT_LAST = N - (NT - 1) * RPT
BN_EPS = 1e-5
BM = 1000

def _mesh():
    return plsc.VectorSubcoreMesh(core_axis_name="c", subcore_axis_name="s")


NBUF = 2
HKCH = KCH // 2
HROUNDS = HKCH // NBUF


def _seg_body(xlo, xhi, src3, dst3, zlo, zhi, srcv, dstv, rows, aggsh, *sems):
    c = lax.axis_index("c")
    s = lax.axis_index("s")
    base = s * RPT
    last = (NT - 1) * RPT
    gsems = sems[:NBUF]
    ssems = sems[NBUF:]

    def run(xh, zh):
        @pl.when(s < NT - 1)
        def _():
            pltpu.sync_copy(xh.at[pl.ds(base, RPT)], aggsh.at[pl.ds(base, RPT)])

        @pl.when(s == NT - 1)
        def _():
            pltpu.sync_copy(xh.at[pl.ds(last, RPT_LAST)],
                            aggsh.at[pl.ds(last, RPT_LAST)])

        plsc.subcore_barrier()

        for hf in range(2):
            pltpu.sync_copy(src3.at[s].at[pl.ds(hf * HKCH, HKCH)], srcv)
            pltpu.sync_copy(dst3.at[s].at[pl.ds(hf * HKCH, HKCH)], dstv)

            for b in range(NBUF):
                pltpu.async_copy(xh.at[srcv.at[b]], rows.at[b], gsems[b])

            @pl.loop(0, HROUNDS)
            def _(j):
                c0 = j * NBUF
                for b in range(NBUF):
                    ce = c0 + b
                    pltpu.make_async_copy(xh.at[srcv.at[ce]], rows.at[b],
                                          gsems[b]).wait()
                    pltpu.async_copy(rows.at[b], aggsh.at[dstv.at[ce]],
                                     ssems[b], add=True)

                @pl.when(j < HROUNDS - 1)
                def _():
                    for b in range(NBUF):
                        ce = c0 + b
                        pltpu.make_async_copy(rows.at[b],
                                              aggsh.at[dstv.at[ce]],
                                              ssems[b]).wait()
                        pltpu.async_copy(xh.at[srcv.at[ce + NBUF]], rows.at[b],
                                         gsems[b])

            for b in range(NBUF):
                ce = HKCH - NBUF + b
                pltpu.make_async_copy(rows.at[b], aggsh.at[dstv.at[ce]],
                                      ssems[b]).wait()

        plsc.subcore_barrier()

        @pl.when(s < NT - 1)
        def _():
            pltpu.sync_copy(aggsh.at[pl.ds(base, RPT)], zh.at[pl.ds(base, RPT)])

        @pl.when(s == NT - 1)
        def _():
            pltpu.sync_copy(aggsh.at[pl.ds(last, RPT_LAST)],
                            zh.at[pl.ds(last, RPT_LAST)])

    @pl.when(c == 0)
    def _():
        run(xlo, zlo)

    @pl.when(c == 1)
    def _():
        run(xhi, zhi)


def _segment_sc(xlo, xhi, src3, dst3):
    f = pl.kernel(
        _seg_body,
        out_type=[jax.ShapeDtypeStruct((N, HALF), jnp.float32),
                  jax.ShapeDtypeStruct((N, HALF), jnp.float32)],
        mesh=_mesh(),
        scratch_types=[
            pltpu.VMEM((HKCH, CHUNK), jnp.int32),
            pltpu.VMEM((HKCH, CHUNK), jnp.int32),
            pltpu.VMEM((NBUF, CHUNK, HALF), jnp.float32),
            pltpu.VMEM_SHARED((N_PAD, HALF), jnp.float32),
        ] + [pltpu.SemaphoreType.DMA] * (2 * NBUF),
    )
    return f(xlo, xhi, src3, dst3)


def _dot(a, b):
    return jnp.dot(a, b, preferred_element_type=jnp.float32,
                   precision=lax.Precision.DEFAULT)


def _mlp_body(zlo, zhi, wa, ba, wb, bb, h_out, stats):
    i = pl.program_id(0)
    u = _dot(zlo[...], wa[0:HALF, :]) + _dot(zhi[...], wa[HALF:COLS, :])
    u = jnp.maximum(u + ba[...], 0.0)
    h = _dot(u, wb[...]) + bb[...]
    h = jnp.maximum(h, 0.0)
    h_out[...] = h

    @pl.when(i == 0)
    def _():
        stats[...] = jnp.zeros_like(stats)

    stats[0:1, :] += jnp.sum(h, axis=0, keepdims=True)
    stats[1:2, :] += jnp.sum(h * h, axis=0, keepdims=True)


def _mlp(zlo, zhi, wa, ba, wb, bb, wout):
    return pl.pallas_call(
        _mlp_body,
        grid=(N // BM,),
        in_specs=[
            pl.BlockSpec((BM, HALF), lambda i: (i, 0)),
            pl.BlockSpec((BM, HALF), lambda i: (i, 0)),
            pl.BlockSpec((COLS, COLS), lambda i: (0, 0)),
            pl.BlockSpec((1, COLS), lambda i: (0, 0)),
            pl.BlockSpec((COLS, wout), lambda i: (0, 0)),
            pl.BlockSpec((1, wout), lambda i: (0, 0)),
        ],
        out_specs=[
            pl.BlockSpec((BM, wout), lambda i: (i, 0)),
            pl.BlockSpec((8, wout), lambda i: (0, 0)),
        ],
        out_shape=[jax.ShapeDtypeStruct((N, wout), jnp.float32),
                   jax.ShapeDtypeStruct((8, wout), jnp.float32)],
    )(zlo, zhi, wa, ba, wb, bb)


def _norm_split_body(h, sref, tref, olo, ohi):
    v = h[...] * sref[...] + tref[...]
    olo[...] = v[:, 0:HALF]
    ohi[...] = v[:, HALF:COLS]


def _norm_split(h, sv, tv):
    return pl.pallas_call(
        _norm_split_body,
        grid=(N // BM,),
        in_specs=[
            pl.BlockSpec((BM, COLS), lambda i: (i, 0)),
            pl.BlockSpec((1, COLS), lambda i: (0, 0)),
            pl.BlockSpec((1, COLS), lambda i: (0, 0)),
        ],
        out_specs=[
            pl.BlockSpec((BM, HALF), lambda i: (i, 0)),
            pl.BlockSpec((BM, HALF), lambda i: (i, 0)),
        ],
        out_shape=[jax.ShapeDtypeStruct((N, HALF), jnp.float32),
                   jax.ShapeDtypeStruct((N, HALF), jnp.float32)],
    )(h, sv, tv)


def _norm_body(h, sref, tref, out):
    out[...] = h[...] * sref[...] + tref[...]


def _norm(h, sv, tv):
    return pl.pallas_call(
        _norm_body,
        grid=(N // BM,),
        in_specs=[
            pl.BlockSpec((BM, HALF), lambda i: (i, 0)),
            pl.BlockSpec((1, HALF), lambda i: (0, 0)),
            pl.BlockSpec((1, HALF), lambda i: (0, 0)),
        ],
        out_specs=pl.BlockSpec((BM, HALF), lambda i: (i, 0)),
        out_shape=jax.ShapeDtypeStruct((N, HALF), jnp.float32),
    )(h, sv, tv)


def _bn_affine(stats, gamma, beta):
    m = stats[0] / N
    v = stats[1] / N - m * m
    s = gamma * lax.rsqrt(v + BN_EPS)
    t = beta - m * s
    return s[None, :], t[None, :]


def kernel(x, edge_index, W1a, b1a, W1b, b1b, g1, beta1,
           W2a, b2a, W2b, b2b, g2, beta2):
    x = x.astype(jnp.float32)
    src = edge_index[0].astype(jnp.int32)
    dst = edge_index[1].astype(jnp.int32)
    pad = E_PAD - E
    src3 = jnp.concatenate([src, jnp.zeros((pad,), jnp.int32)]).reshape(
        NT, KCH, CHUNK)
    dst3 = jnp.concatenate([dst, jnp.full((pad,), N, jnp.int32)]).reshape(
        NT, KCH, CHUNK)

    xlo = x[:, :HALF]
    xhi = x[:, HALF:]

    z1lo, z1hi = _segment_sc(xlo, xhi, src3, dst3)
    h1, st1 = _mlp(z1lo, z1hi, W1a, b1a[None, :], W1b, b1b[None, :], COLS)
    s1, t1 = _bn_affine(st1, g1, beta1)
    h1lo, h1hi = _norm_split(h1, s1, t1)

    z2lo, z2hi = _segment_sc(h1lo, h1hi, src3, dst3)
    w2bp = jnp.pad(W2b, ((0, 0), (0, HALF - OD)))
    b2bp = jnp.pad(b2b, (0, HALF - OD))
    o, st2 = _mlp(z2lo, z2hi, W2a, b2a[None, :], w2bp, b2bp[None, :], HALF)
    g2p = jnp.pad(g2, (0, HALF - OD))
    beta2p = jnp.pad(beta2, (0, HALF - OD))
    s2, t2 = _bn_affine(st2, g2p, beta2p)
    outp = _norm(o, s2, t2)
    return outp[:, :OD]


# speedup vs baseline: 3.3404x; 1.0339x over previous
import functools

import jax
import jax.numpy as jnp
from jax import lax
from jax.experimental import pallas as pl
from jax.experimental.pallas import tpu as pltpu
from jax.experimental.pallas import tpu_sc as plsc

N = 10000
E = 160000
COLS = 256
HALF = 128
OD = 2
NT = 16
CHUNK = 64
KCH = 160
E_PAD = NT * KCH * CHUNK
N_PAD = 10008
RPT = 632
RPT_LAST = N - (NT - 1) * RPT
BN_EPS = 1e-5
BM = 1000

def _mesh():
    return plsc.VectorSubcoreMesh(core_axis_name="c", subcore_axis_name="s")


NBUF = 4
AHEAD = 2
NSLAB = 4
HKCH = KCH // NSLAB


def _seg_body(xlo, xhi, src3, dst3, zlo, zhi, srcv, dstv, rows, aggsh, *sems):
    c = lax.axis_index("c")
    s = lax.axis_index("s")
    base = s * RPT
    last = (NT - 1) * RPT
    gsems = sems[:NBUF]
    ssems = sems[NBUF:]

    def run(xh, zh):
        @pl.when(s < NT - 1)
        def _():
            pltpu.sync_copy(xh.at[pl.ds(base, RPT)], aggsh.at[pl.ds(base, RPT)])

        @pl.when(s == NT - 1)
        def _():
            pltpu.sync_copy(xh.at[pl.ds(last, RPT_LAST)],
                            aggsh.at[pl.ds(last, RPT_LAST)])

        plsc.subcore_barrier()

        for hf in range(NSLAB):
            pltpu.sync_copy(src3.at[s].at[pl.ds(hf * HKCH, HKCH)], srcv)
            pltpu.sync_copy(dst3.at[s].at[pl.ds(hf * HKCH, HKCH)], dstv)

            for ce in range(AHEAD):
                pltpu.async_copy(xh.at[srcv.at[ce]], rows.at[ce % NBUF],
                                 gsems[ce % NBUF])

            for ce in range(HKCH):
                b = ce % NBUF
                pltpu.make_async_copy(xh.at[srcv.at[ce]], rows.at[b],
                                      gsems[b]).wait()
                pltpu.async_copy(rows.at[b], aggsh.at[dstv.at[ce]],
                                 ssems[b], add=True)
                nc = ce + AHEAD
                if nc < HKCH:
                    nb = nc % NBUF
                    if nc >= NBUF:
                        pltpu.make_async_copy(rows.at[nb],
                                              aggsh.at[dstv.at[nc - NBUF]],
                                              ssems[nb]).wait()
                    pltpu.async_copy(xh.at[srcv.at[nc]], rows.at[nb],
                                     gsems[nb])

            for ce in range(HKCH - NBUF, HKCH):
                b = ce % NBUF
                pltpu.make_async_copy(rows.at[b], aggsh.at[dstv.at[ce]],
                                      ssems[b]).wait()

        plsc.subcore_barrier()

        @pl.when(s < NT - 1)
        def _():
            pltpu.sync_copy(aggsh.at[pl.ds(base, RPT)], zh.at[pl.ds(base, RPT)])

        @pl.when(s == NT - 1)
        def _():
            pltpu.sync_copy(aggsh.at[pl.ds(last, RPT_LAST)],
                            zh.at[pl.ds(last, RPT_LAST)])

    @pl.when(c == 0)
    def _():
        run(xlo, zlo)

    @pl.when(c == 1)
    def _():
        run(xhi, zhi)


def _segment_sc(xlo, xhi, src3, dst3):
    f = pl.kernel(
        _seg_body,
        out_type=[jax.ShapeDtypeStruct((N, HALF), jnp.float32),
                  jax.ShapeDtypeStruct((N, HALF), jnp.float32)],
        mesh=_mesh(),
        scratch_types=[
            pltpu.VMEM((HKCH, CHUNK), jnp.int32),
            pltpu.VMEM((HKCH, CHUNK), jnp.int32),
            pltpu.VMEM((NBUF, CHUNK, HALF), jnp.float32),
            pltpu.VMEM_SHARED((N_PAD, HALF), jnp.float32),
        ] + [pltpu.SemaphoreType.DMA] * (2 * NBUF),
    )
    return f(xlo, xhi, src3, dst3)


def _dot(a, b):
    return jnp.dot(a, b, preferred_element_type=jnp.float32,
                   precision=lax.Precision.DEFAULT)


def _mlp_body(zlo, zhi, wa, ba, wb, bb, h_out, stats):
    i = pl.program_id(0)
    u = _dot(zlo[...], wa[0:HALF, :]) + _dot(zhi[...], wa[HALF:COLS, :])
    u = jnp.maximum(u + ba[...], 0.0)
    h = _dot(u, wb[...]) + bb[...]
    h = jnp.maximum(h, 0.0)
    h_out[...] = h

    @pl.when(i == 0)
    def _():
        stats[...] = jnp.zeros_like(stats)

    stats[0:1, :] += jnp.sum(h, axis=0, keepdims=True)
    stats[1:2, :] += jnp.sum(h * h, axis=0, keepdims=True)


def _mlp(zlo, zhi, wa, ba, wb, bb, wout):
    return pl.pallas_call(
        _mlp_body,
        grid=(N // BM,),
        in_specs=[
            pl.BlockSpec((BM, HALF), lambda i: (i, 0)),
            pl.BlockSpec((BM, HALF), lambda i: (i, 0)),
            pl.BlockSpec((COLS, COLS), lambda i: (0, 0)),
            pl.BlockSpec((1, COLS), lambda i: (0, 0)),
            pl.BlockSpec((COLS, wout), lambda i: (0, 0)),
            pl.BlockSpec((1, wout), lambda i: (0, 0)),
        ],
        out_specs=[
            pl.BlockSpec((BM, wout), lambda i: (i, 0)),
            pl.BlockSpec((8, wout), lambda i: (0, 0)),
        ],
        out_shape=[jax.ShapeDtypeStruct((N, wout), jnp.float32),
                   jax.ShapeDtypeStruct((8, wout), jnp.float32)],
    )(zlo, zhi, wa, ba, wb, bb)


def _norm_split_body(h, sref, tref, olo, ohi):
    v = h[...] * sref[...] + tref[...]
    olo[...] = v[:, 0:HALF]
    ohi[...] = v[:, HALF:COLS]


def _norm_split(h, sv, tv):
    return pl.pallas_call(
        _norm_split_body,
        grid=(N // BM,),
        in_specs=[
            pl.BlockSpec((BM, COLS), lambda i: (i, 0)),
            pl.BlockSpec((1, COLS), lambda i: (0, 0)),
            pl.BlockSpec((1, COLS), lambda i: (0, 0)),
        ],
        out_specs=[
            pl.BlockSpec((BM, HALF), lambda i: (i, 0)),
            pl.BlockSpec((BM, HALF), lambda i: (i, 0)),
        ],
        out_shape=[jax.ShapeDtypeStruct((N, HALF), jnp.float32),
                   jax.ShapeDtypeStruct((N, HALF), jnp.float32)],
    )(h, sv, tv)


def _norm_body(h, sref, tref, out):
    out[...] = h[...] * sref[...] + tref[...]


def _norm(h, sv, tv):
    return pl.pallas_call(
        _norm_body,
        grid=(N // BM,),
        in_specs=[
            pl.BlockSpec((BM, HALF), lambda i: (i, 0)),
            pl.BlockSpec((1, HALF), lambda i: (0, 0)),
            pl.BlockSpec((1, HALF), lambda i: (0, 0)),
        ],
        out_specs=pl.BlockSpec((BM, HALF), lambda i: (i, 0)),
        out_shape=jax.ShapeDtypeStruct((N, HALF), jnp.float32),
    )(h, sv, tv)


def _bn_affine(stats, gamma, beta):
    m = stats[0] / N
    v = stats[1] / N - m * m
    s = gamma * lax.rsqrt(v + BN_EPS)
    t = beta - m * s
    return s[None, :], t[None, :]


def kernel(x, edge_index, W1a, b1a, W1b, b1b, g1, beta1,
           W2a, b2a, W2b, b2b, g2, beta2):
    x = x.astype(jnp.float32)
    src = edge_index[0].astype(jnp.int32)
    dst = edge_index[1].astype(jnp.int32)
    pad = E_PAD - E
    src3 = jnp.concatenate([src, jnp.zeros((pad,), jnp.int32)]).reshape(
        NT, KCH, CHUNK)
    dst3 = jnp.concatenate([dst, jnp.full((pad,), N, jnp.int32)]).reshape(
        NT, KCH, CHUNK)

    xlo = x[:, :HALF]
    xhi = x[:, HALF:]

    z1lo, z1hi = _segment_sc(xlo, xhi, src3, dst3)
    h1, st1 = _mlp(z1lo, z1hi, W1a, b1a[None, :], W1b, b1b[None, :], COLS)
    s1, t1 = _bn_affine(st1, g1, beta1)
    h1lo, h1hi = _norm_split(h1, s1, t1)

    z2lo, z2hi = _segment_sc(h1lo, h1hi, src3, dst3)
    w2bp = jnp.pad(W2b, ((0, 0), (0, HALF - OD)))
    b2bp = jnp.pad(b2b, (0, HALF - OD))
    o, st2 = _mlp(z2lo, z2hi, W2a, b2a[None, :], w2bp, b2bp[None, :], HALF)
    g2p = jnp.pad(g2, (0, HALF - OD))
    beta2p = jnp.pad(beta2, (0, HALF - OD))
    s2, t2 = _bn_affine(st2, g2p, beta2p)
    outp = _norm(o, s2, t2)
    return outp[:, :OD]
